# baseline (speedup 1.0000x reference)
import functools

import numpy as np
import jax
import jax.numpy as jnp
from jax import lax
from jax.experimental import pallas as pl
from jax.experimental.pallas import tpu as pltpu
from jax.experimental.pallas import tpu_sc as plsc

N = 16384
N_PAIRS = 32
LANES = 16
NCHUNKS = N // LANES
NBINS = 128
INV_TOTAL = 1.0 / (N * N_PAIRS)

_U32 = np.uint32


def _threefry2x32(k0, k1, x0, x1):
    x0 = x0.astype(_U32).copy()
    x1 = x1.astype(_U32).copy()
    ks0 = _U32(k0)
    ks1 = _U32(k1)
    ks2 = _U32(np.uint32(0x1BD11BDA) ^ ks0 ^ ks1)
    ks = [ks0, ks1, ks2]
    rotations = [(13, 15, 26, 6), (17, 29, 16, 24)]
    with np.errstate(over="ignore"):
        x0 = (x0 + ks0).astype(_U32)
        x1 = (x1 + ks1).astype(_U32)
        for i in range(5):
            for r in rotations[i % 2]:
                x0 = (x0 + x1).astype(_U32)
                x1 = ((x1 << _U32(r)) | (x1 >> _U32(32 - r))).astype(_U32)
                x1 = (x1 ^ x0).astype(_U32)
            x0 = (x0 + ks[(i + 1) % 3]).astype(_U32)
            x1 = (x1 + ks[(i + 2) % 3] + _U32(i + 1)).astype(_U32)
    return x0, x1


def _sigma_const():
    rows = []
    lo = np.zeros(N, dtype=_U32)
    counts = np.arange(N, dtype=_U32)
    for n in range(N_PAIRS):
        a, b = _threefry2x32(0, 42, np.array([0], _U32), np.array([n], _U32))
        o0, o1 = _threefry2x32(a[0], b[0], lo, counts)
        bits = o0 ^ o1
        r = ((bits >> _U32(9)) | _U32(0x3F800000)).view(np.float32) - np.float32(1.0)
        r = np.maximum(np.float32(0.0), r)
        rows.append(np.argsort(r, kind="stable").astype(np.int32))
    return np.stack(rows)


_SIGMA = _sigma_const()


@functools.cache
def _build_rankloss_sc():
    return functools.partial(
        pl.kernel,
        mesh=plsc.VectorSubcoreMesh(core_axis_name="c", subcore_axis_name="s"),
        compiler_params=pltpu.CompilerParams(needs_layout_passes=False),
        out_type=jax.ShapeDtypeStruct((N_PAIRS, LANES), jnp.float32),
        scratch_types=[
            pltpu.VMEM((N,), jnp.float32),
            pltpu.VMEM((N,), jnp.float32),
            pltpu.VMEM((N + 2 * LANES,), jnp.int32),
            pltpu.VMEM((N + 2 * LANES,), jnp.int32),
            pltpu.VMEM((N,), jnp.int32),
            pltpu.VMEM((N,), jnp.int32),
            pltpu.VMEM((NBINS,), jnp.int32),
            pltpu.VMEM((NBINS,), jnp.int32),
            pltpu.VMEM((NBINS,), jnp.int32),
            pltpu.VMEM((LANES,), jnp.float32),
            pltpu.SemaphoreType.DMA,
        ],
    )(_rankloss_sc)


def _rankloss_sc(pred_hbm, count_hbm, groups_hbm, sigma_hbm, out_hbm,
                 pred_v, count_v, groups_v, sig_v, a_v, b_v,
                 hist_v, cur2_v, curs_v, out_v, dma_sem):
    wid = lax.axis_index("c") * 16 + lax.axis_index("s")

    early_copies = [
        pltpu.async_copy(groups_hbm, groups_v.at[pl.ds(0, N)], dma_sem),
        pltpu.async_copy(sigma_hbm.at[wid], sig_v.at[pl.ds(0, N)], dma_sem),
    ]
    late_copies = [
        pltpu.async_copy(pred_hbm, pred_v, dma_sem),
        pltpu.async_copy(count_hbm, count_v, dma_sem),
    ]

    lane = lax.iota(jnp.int32, LANES)
    zeros = jnp.zeros((LANES,), jnp.int32)

    @plsc.parallel_loop(0, NBINS // LANES, unroll=4)
    def zero_body(i):
        hist_v[pl.ds(i * LANES, LANES)] = zeros

    for c in early_copies:
        c.wait()
    groups_v[pl.ds(N, LANES)] = zeros
    groups_v[pl.ds(N + LANES, LANES)] = zeros
    sig_v[pl.ds(N, LANES)] = zeros
    sig_v[pl.ds(N + LANES, LANES)] = zeros

    @plsc.parallel_loop(0, NCHUNKS, unroll=8)
    def hist_body(i):
        g = groups_v[pl.ds(i * LANES, LANES)]
        occ, last = plsc.scan_count(g)
        plsc.addupdate_scatter(hist_v, [g], occ, mask=last)

    def prefix_body(j, carry):
        off = j * LANES
        row = hist_v[pl.ds(off, LANES)]
        incl = plsc.cumsum(row)
        excl = incl - row + carry
        cur2_v[pl.ds(off, LANES)] = excl
        curs_v[pl.ds(off, LANES)] = excl
        return carry + jnp.sum(row)
    lax.fori_loop(0, NBINS // LANES, prefix_body, jnp.int32(0))

    def prefetch(i):
        off = i * LANES
        g = groups_v[pl.ds(off, LANES)]
        occ2, last2 = plsc.scan_count(g)
        v = sig_v[pl.ds(off, LANES)]
        gs = plsc.load_gather(groups_v, [v])
        occs, lasts = plsc.scan_count(gs)
        return (g, occ2, last2, v, gs, occs, lasts)

    def build_body(i, carry):
        (g, occ2, last2, v, gs, occs, lasts), nxt = carry
        base2 = plsc.load_gather(cur2_v, [g])
        bases = plsc.load_gather(curs_v, [gs])
        plsc.store_scatter(cur2_v, [g], base2 + occ2, mask=last2)
        plsc.store_scatter(a_v, [base2 + occ2 - 1], i * LANES + lane)
        plsc.store_scatter(curs_v, [gs], bases + occs, mask=lasts)
        plsc.store_scatter(b_v, [bases + occs - 1], v)
        return nxt, prefetch(i + 2)
    lax.fori_loop(0, NCHUNKS, build_body, (prefetch(0), prefetch(1)))

    for c in late_copies:
        c.wait()

    @plsc.parallel_loop(0, NCHUNKS, unroll=8,
                        carry=jnp.zeros((LANES,), jnp.float32))
    def acc_body(i, acc):
        off = i * LANES
        u = a_v[pl.ds(off, LANES)]
        v = b_v[pl.ds(off, LANES)]
        pu = plsc.load_gather(pred_v, [u])
        pv = plsc.load_gather(pred_v, [v])
        cu = plsc.load_gather(count_v, [u])
        cv = plsc.load_gather(count_v, [v])
        d = pu - pv
        return acc + jnp.maximum(jnp.where(cu > cv, -d, d), 0.0)
    acc = acc_body

    out_v[...] = acc * INV_TOTAL
    pltpu.sync_copy(out_v, out_hbm.at[wid])


def kernel(pred, count, groups):
    sigma = jnp.asarray(_SIGMA)
    partials = _build_rankloss_sc()(pred, count, groups, sigma)
    return jnp.sum(partials)


# speedup vs baseline: 1.1387x; 1.1387x over previous
import functools

import numpy as np
import jax
import jax.numpy as jnp
from jax import lax
from jax.experimental import pallas as pl
from jax.experimental.pallas import tpu as pltpu
from jax.experimental.pallas import tpu_sc as plsc

N = 16384
N_PAIRS = 32
LANES = 16
SLICE = N // LANES
NBINS = 128
UNROLL = 4
INV_TOTAL = 1.0 / (N * N_PAIRS)

_U32 = np.uint32


def _threefry2x32(k0, k1, x0, x1):
    x0 = x0.astype(_U32).copy()
    x1 = x1.astype(_U32).copy()
    ks0 = _U32(k0)
    ks1 = _U32(k1)
    ks2 = _U32(np.uint32(0x1BD11BDA) ^ ks0 ^ ks1)
    ks = [ks0, ks1, ks2]
    rotations = [(13, 15, 26, 6), (17, 29, 16, 24)]
    with np.errstate(over="ignore"):
        x0 = (x0 + ks0).astype(_U32)
        x1 = (x1 + ks1).astype(_U32)
        for i in range(5):
            for r in rotations[i % 2]:
                x0 = (x0 + x1).astype(_U32)
                x1 = ((x1 << _U32(r)) | (x1 >> _U32(32 - r))).astype(_U32)
                x1 = (x1 ^ x0).astype(_U32)
            x0 = (x0 + ks[(i + 1) % 3]).astype(_U32)
            x1 = (x1 + ks[(i + 2) % 3] + _U32(i + 1)).astype(_U32)
    return x0, x1


def _sigma_const():
    rows = []
    lo = np.zeros(N, dtype=_U32)
    counts = np.arange(N, dtype=_U32)
    for n in range(N_PAIRS):
        a, b = _threefry2x32(0, 42, np.array([0], _U32), np.array([n], _U32))
        o0, o1 = _threefry2x32(a[0], b[0], lo, counts)
        bits = o0 ^ o1
        r = ((bits >> _U32(9)) | _U32(0x3F800000)).view(np.float32) - np.float32(1.0)
        r = np.maximum(np.float32(0.0), r)
        sig = np.argsort(r, kind="stable").astype(np.int32)
        rows.append(sig.reshape(LANES, SLICE).T.reshape(N))
    return np.stack(rows)


_SIGMA_T = _sigma_const()


@functools.cache
def _build_rankloss_sc():
    return functools.partial(
        pl.kernel,
        mesh=plsc.VectorSubcoreMesh(core_axis_name="c", subcore_axis_name="s"),
        compiler_params=pltpu.CompilerParams(needs_layout_passes=False),
        out_type=jax.ShapeDtypeStruct((N_PAIRS, LANES), jnp.float32),
        scratch_types=[
            pltpu.VMEM((N,), jnp.float32),
            pltpu.VMEM((N,), jnp.float32),
            pltpu.VMEM((N,), jnp.int32),
            pltpu.VMEM((N + 2 * LANES,), jnp.int32),
            pltpu.VMEM((N + 2 * LANES,), jnp.int32),
            pltpu.VMEM((N,), jnp.int32),
            pltpu.VMEM((N,), jnp.int32),
            pltpu.VMEM((NBINS * LANES,), jnp.int32),
            pltpu.VMEM((NBINS * LANES,), jnp.int32),
            pltpu.VMEM((LANES,), jnp.float32),
            pltpu.SemaphoreType.DMA,
        ],
    )(_rankloss_sc)


def _rankloss_sc(pred_hbm, count_hbm, groups_hbm, sigma_hbm, out_hbm,
                 pred_v, count_v, groups_v, sigt_v, gt_v, a_v, b_v,
                 cnt2_v, cnts_v, out_v, dma_sem):
    wid = lax.axis_index("c") * 16 + lax.axis_index("s")

    early_copies = [
        pltpu.async_copy(groups_hbm, groups_v, dma_sem),
        pltpu.async_copy(sigma_hbm.at[wid], sigt_v.at[pl.ds(0, N)], dma_sem),
    ]
    late_copies = [
        pltpu.async_copy(pred_hbm, pred_v, dma_sem),
        pltpu.async_copy(count_hbm, count_v, dma_sem),
    ]

    lane = lax.iota(jnp.int32, LANES)
    lane_base = lane * SLICE
    lane16 = lane
    zeros = jnp.zeros((LANES,), jnp.int32)
    ones = jnp.ones((LANES,), jnp.int32)

    @plsc.parallel_loop(0, NBINS, unroll=4)
    def zero_body(i):
        off = i * LANES
        cnt2_v[pl.ds(off, LANES)] = zeros
        cnts_v[pl.ds(off, LANES)] = zeros

    for c in early_copies:
        c.wait()

    @plsc.parallel_loop(0, SLICE, unroll=8)
    def hist_body(s):
        off = s * LANES
        vs = sigt_v[pl.ds(off, LANES)]
        gs = plsc.load_gather(groups_v, [vs])
        plsc.addupdate_scatter(cnts_v, [gs * LANES + lane16], ones)
        gi = plsc.load_gather(groups_v, [lane_base + s])
        gt_v[pl.ds(off, LANES)] = gi
        plsc.addupdate_scatter(cnt2_v, [gi * LANES + lane16], ones)

    def prefix_body(g, carry):
        off = g * LANES
        row2 = cnt2_v[pl.ds(off, LANES)]
        rows = cnts_v[pl.ds(off, LANES)]
        incl2 = plsc.cumsum(row2)
        incls = plsc.cumsum(rows)
        cnt2_v[pl.ds(off, LANES)] = incl2 - row2 + carry
        cnts_v[pl.ds(off, LANES)] = incls - rows + carry
        return carry + jnp.sum(row2)
    lax.fori_loop(0, NBINS, prefix_body, jnp.int32(0))

    gt_v[pl.ds(N, LANES)] = zeros
    gt_v[pl.ds(N + LANES, LANES)] = zeros
    sigt_v[pl.ds(N, LANES)] = zeros
    sigt_v[pl.ds(N + LANES, LANES)] = zeros

    gi0 = gt_v[pl.ds(0, LANES)]
    vs0 = sigt_v[pl.ds(0, LANES)]
    gs0 = plsc.load_gather(groups_v, [vs0])
    carry0 = (gi0 * LANES + lane16, vs0, gs0 * LANES + lane16,
              gt_v[pl.ds(LANES, LANES)], sigt_v[pl.ds(LANES, LANES)])

    def build_step(s, carry):
        addr2, vs, addrs, gi1, vs1 = carry
        gs1 = plsc.load_gather(groups_v, [vs1])
        pos2 = plsc.load_gather(cnt2_v, [addr2])
        poss = plsc.load_gather(cnts_v, [addrs])
        plsc.store_scatter(cnt2_v, [addr2], pos2 + 1)
        plsc.store_scatter(cnts_v, [addrs], poss + 1)
        plsc.store_scatter(a_v, [pos2], lane_base + s)
        plsc.store_scatter(b_v, [poss], vs)
        off2 = (s + 2) * LANES
        return (gi1 * LANES + lane16, vs1, gs1 * LANES + lane16,
                gt_v[pl.ds(off2, LANES)], sigt_v[pl.ds(off2, LANES)])

    def build_body(i, carry):
        carry = build_step(i * 2, carry)
        return build_step(i * 2 + 1, carry)
    lax.fori_loop(0, SLICE // 2, build_body, carry0)

    for c in late_copies:
        c.wait()

    @plsc.parallel_loop(0, SLICE, unroll=8,
                        carry=jnp.zeros((LANES,), jnp.float32))
    def acc_body(s, acc):
        off = s * LANES
        u = a_v[pl.ds(off, LANES)]
        v = b_v[pl.ds(off, LANES)]
        pu = plsc.load_gather(pred_v, [u])
        pv = plsc.load_gather(pred_v, [v])
        cu = plsc.load_gather(count_v, [u])
        cv = plsc.load_gather(count_v, [v])
        d = pu - pv
        return acc + jnp.maximum(jnp.where(cu > cv, -d, d), 0.0)
    acc = acc_body

    out_v[...] = acc * INV_TOTAL
    pltpu.sync_copy(out_v, out_hbm.at[wid])


def kernel(pred, count, groups):
    sigma = jnp.asarray(_SIGMA_T)
    partials = _build_rankloss_sc()(pred, count, groups, sigma)
    return jnp.sum(partials)


# speedup vs baseline: 1.1505x; 1.0104x over previous
import functools

import numpy as np
import jax
import jax.numpy as jnp
from jax import lax
from jax.experimental import pallas as pl
from jax.experimental.pallas import tpu as pltpu
from jax.experimental.pallas import tpu_sc as plsc

N = 16384
N_PAIRS = 32
LANES = 16
SLICE = N // LANES
NBINS = 128
UNROLL = 4
INV_TOTAL = 1.0 / (N * N_PAIRS)

_U32 = np.uint32


def _threefry2x32(k0, k1, x0, x1):
    x0 = x0.astype(_U32).copy()
    x1 = x1.astype(_U32).copy()
    ks0 = _U32(k0)
    ks1 = _U32(k1)
    ks2 = _U32(np.uint32(0x1BD11BDA) ^ ks0 ^ ks1)
    ks = [ks0, ks1, ks2]
    rotations = [(13, 15, 26, 6), (17, 29, 16, 24)]
    with np.errstate(over="ignore"):
        x0 = (x0 + ks0).astype(_U32)
        x1 = (x1 + ks1).astype(_U32)
        for i in range(5):
            for r in rotations[i % 2]:
                x0 = (x0 + x1).astype(_U32)
                x1 = ((x1 << _U32(r)) | (x1 >> _U32(32 - r))).astype(_U32)
                x1 = (x1 ^ x0).astype(_U32)
            x0 = (x0 + ks[(i + 1) % 3]).astype(_U32)
            x1 = (x1 + ks[(i + 2) % 3] + _U32(i + 1)).astype(_U32)
    return x0, x1


def _sigma_const():
    rows = []
    lo = np.zeros(N, dtype=_U32)
    counts = np.arange(N, dtype=_U32)
    for n in range(N_PAIRS):
        a, b = _threefry2x32(0, 42, np.array([0], _U32), np.array([n], _U32))
        o0, o1 = _threefry2x32(a[0], b[0], lo, counts)
        bits = o0 ^ o1
        r = ((bits >> _U32(9)) | _U32(0x3F800000)).view(np.float32) - np.float32(1.0)
        r = np.maximum(np.float32(0.0), r)
        sig = np.argsort(r, kind="stable").astype(np.int32)
        rows.append(sig.reshape(LANES, SLICE).T.reshape(N))
    return np.stack(rows)


_SIGMA_T = _sigma_const()


@functools.cache
def _build_rankloss_sc():
    return functools.partial(
        pl.kernel,
        mesh=plsc.VectorSubcoreMesh(core_axis_name="c", subcore_axis_name="s"),
        compiler_params=pltpu.CompilerParams(needs_layout_passes=False),
        out_type=jax.ShapeDtypeStruct((N_PAIRS, LANES), jnp.float32),
        scratch_types=[
            pltpu.VMEM((N,), jnp.float32),
            pltpu.VMEM((N,), jnp.float32),
            pltpu.VMEM((N,), jnp.int32),
            pltpu.VMEM((N + 2 * LANES,), jnp.int32),
            pltpu.VMEM((N + 2 * LANES,), jnp.int32),
            pltpu.VMEM((N,), jnp.int32),
            pltpu.VMEM((N,), jnp.int32),
            pltpu.VMEM((NBINS * LANES,), jnp.int32),
            pltpu.VMEM((NBINS * LANES,), jnp.int32),
            pltpu.VMEM((LANES,), jnp.float32),
            pltpu.SemaphoreType.DMA,
        ],
    )(_rankloss_sc)


def _rankloss_sc(pred_hbm, count_hbm, groups_hbm, sigma_hbm, out_hbm,
                 pred_v, count_v, groups_v, sigt_v, gt_v, a_v, b_v,
                 cnt2_v, cnts_v, out_v, dma_sem):
    wid = lax.axis_index("c") * 16 + lax.axis_index("s")

    early_copies = [
        pltpu.async_copy(groups_hbm, groups_v, dma_sem),
        pltpu.async_copy(sigma_hbm.at[wid], sigt_v.at[pl.ds(0, N)], dma_sem),
    ]
    late_copies = [
        pltpu.async_copy(pred_hbm, pred_v, dma_sem),
        pltpu.async_copy(count_hbm, count_v, dma_sem),
    ]

    lane = lax.iota(jnp.int32, LANES)
    lane_base = lane * SLICE
    lane16 = lane
    zeros = jnp.zeros((LANES,), jnp.int32)
    ones = jnp.ones((LANES,), jnp.int32)

    @plsc.parallel_loop(0, NBINS, unroll=4)
    def zero_body(i):
        off = i * LANES
        cnt2_v[pl.ds(off, LANES)] = zeros
        cnts_v[pl.ds(off, LANES)] = zeros

    for c in early_copies:
        c.wait()

    @plsc.parallel_loop(0, SLICE, unroll=8)
    def hist_body(s):
        off = s * LANES
        vs = sigt_v[pl.ds(off, LANES)]
        gs = plsc.load_gather(groups_v, [vs])
        plsc.addupdate_scatter(cnts_v, [gs * LANES + lane16], ones)
        gi = plsc.load_gather(groups_v, [lane_base + s])
        gt_v[pl.ds(off, LANES)] = gi
        plsc.addupdate_scatter(cnt2_v, [gi * LANES + lane16], ones)

    def prefix_body(g, carry):
        off = g * LANES
        row2 = cnt2_v[pl.ds(off, LANES)]
        rows = cnts_v[pl.ds(off, LANES)]
        incl2 = plsc.cumsum(row2)
        incls = plsc.cumsum(rows)
        cnt2_v[pl.ds(off, LANES)] = incl2 - row2 + carry
        cnts_v[pl.ds(off, LANES)] = incls - rows + carry
        return carry + jnp.sum(row2)
    lax.fori_loop(0, NBINS, prefix_body, jnp.int32(0))

    gt_v[pl.ds(N, LANES)] = zeros
    gt_v[pl.ds(N + LANES, LANES)] = zeros
    sigt_v[pl.ds(N, LANES)] = zeros
    sigt_v[pl.ds(N + LANES, LANES)] = zeros

    gi0 = gt_v[pl.ds(0, LANES)]
    vs0 = sigt_v[pl.ds(0, LANES)]
    gs0 = plsc.load_gather(groups_v, [vs0])
    carry0 = (gi0 * LANES + lane16, vs0, gs0 * LANES + lane16,
              gt_v[pl.ds(LANES, LANES)], sigt_v[pl.ds(LANES, LANES)])

    def build_body(s, carry):
        addr2, vs, addrs, gi1, vs1 = carry
        gs1 = plsc.load_gather(groups_v, [vs1])
        pos2 = plsc.load_gather(cnt2_v, [addr2])
        poss = plsc.load_gather(cnts_v, [addrs])
        plsc.store_scatter(cnt2_v, [addr2], pos2 + 1)
        plsc.store_scatter(a_v, [pos2], lane_base + s)
        plsc.store_scatter(cnts_v, [addrs], poss + 1)
        plsc.store_scatter(b_v, [poss], vs)
        off2 = (s + 2) * LANES
        return (gi1 * LANES + lane16, vs1, gs1 * LANES + lane16,
                gt_v[pl.ds(off2, LANES)], sigt_v[pl.ds(off2, LANES)])
    lax.fori_loop(0, SLICE, build_body, carry0)

    for c in late_copies:
        c.wait()

    @plsc.parallel_loop(0, SLICE, unroll=8,
                        carry=jnp.zeros((LANES,), jnp.float32))
    def acc_body(s, acc):
        off = s * LANES
        u = a_v[pl.ds(off, LANES)]
        v = b_v[pl.ds(off, LANES)]
        pu = plsc.load_gather(pred_v, [u])
        pv = plsc.load_gather(pred_v, [v])
        cu = plsc.load_gather(count_v, [u])
        cv = plsc.load_gather(count_v, [v])
        d = pu - pv
        return acc + jnp.maximum(jnp.where(cu > cv, -d, d), 0.0)
    acc = acc_body

    out_v[...] = acc * INV_TOTAL
    pltpu.sync_copy(out_v, out_hbm.at[wid])


def kernel(pred, count, groups):
    sigma = jnp.asarray(_SIGMA_T)
    partials = _build_rankloss_sc()(pred, count, groups, sigma)
    return jnp.sum(partials)
